# Initial kernel scaffold; baseline (speedup 1.0000x reference)
import jax
import jax.numpy as jnp
from jax.experimental import pallas as pl


def kernel(hn, he, edge_index, W1, al1, ar1, b1, W2, al2, ar2, b2, W3, al3, ar3, b3, Wm3, bm3, Wm4, bm4, Wm5, bm5, Wc, bc):
    raise NotImplementedError("write your pallas kernel here")


# speedup vs baseline: 9.3947x; 9.3947x over previous
import jax
import jax.numpy as jnp
from jax import lax
from jax.experimental import pallas as pl
from jax.experimental.pallas import tpu as pltpu
from jax.experimental.pallas import tpu_sc as plsc

_NCORES = 2
_NSUB = 16
_NW = _NCORES * _NSUB
_K = 80


def _mesh():
    return plsc.VectorSubcoreMesh(core_axis_name="c", subcore_axis_name="s")


def _chain(x, dep):
    return lax.optimization_barrier((x, dep))[0]


def _zero_buf(buf, rows, width):
    z = jnp.zeros((16,), jnp.float32)

    def body(k, _):
        for j in range(width // 16):
            buf[k, pl.ds(j * 16, 16)] = z
        return 0

    lax.fori_loop(0, rows, body, 0)


def _acc_slices(n):
    zbase = (n // (8 * _NSUB)) * 8
    tail = n - zbase * _NSUB
    return zbase, tail


def _zero_acc(zbuf_v, acc, sid, n):
    zbase, tail = _acc_slices(n)
    zstart = pl.multiple_of(sid * zbase, 8)
    _zero_slice(zbuf_v, acc, zstart, zbase)
    if tail:
        @pl.when(sid == _NSUB - 1)
        def _():
            _zero_slice(zbuf_v, acc, zbase * _NSUB, tail)


def _writeout_acc(acc, out_slice_fn, sid, n):
    zbase, tail = _acc_slices(n)
    zstart = pl.multiple_of(sid * zbase, 8)
    pltpu.sync_copy(acc.at[pl.ds(zstart, zbase)], out_slice_fn(zstart, zbase))
    if tail:
        @pl.when(sid == _NSUB - 1)
        def _():
            pltpu.sync_copy(acc.at[pl.ds(zbase * _NSUB, tail)],
                            out_slice_fn(zbase * _NSUB, tail))


def _zero_slice(zbuf_v, acc, start, rows):
    nfull = rows // _K
    for t in range(nfull):
        pltpu.sync_copy(zbuf_v, acc.at[pl.ds(start + t * _K, _K)])
    rem = rows - nfull * _K
    if rem:
        pltpu.sync_copy(zbuf_v.at[pl.ds(0, rem)],
                        acc.at[pl.ds(start + nfull * _K, rem)])


def _sc_gather_scatter(n, e, heads, scaled):
    ept = e // _NW
    nchunk = ept // _K
    rps = n // _NSUB

    def body(table, src, dst, w, out, src_v, dst_v, gidx_v, didx_v, rows_v,
             zbuf_v, w_v, acc):
        cid = lax.axis_index("c")
        sid = lax.axis_index("s")
        base_e = (cid * _NSUB + sid) * ept
        pltpu.sync_copy(src.at[pl.ds(base_e, ept)], src_v)
        pltpu.sync_copy(dst.at[pl.ds(base_e, ept)], dst_v)
        _zero_buf(zbuf_v, _K, 128)
        for h in range(heads):
            _zero_acc(zbuf_v, acc, sid, n)
            plsc.subcore_barrier()

            def chunk(j, _):
                base = j * _K
                for g in range(_K // 16):
                    s16 = src_v[pl.ds(base + g * 16, 16)]
                    gidx_v[pl.ds(g * 16, 16)] = s16 + (h * n)
                    didx_v[pl.ds(g * 16, 16)] = dst_v[pl.ds(base + g * 16, 16)]
                pltpu.sync_copy(table.at[gidx_v], rows_v)
                if scaled:
                    pltpu.sync_copy(
                        w.at[pl.ds((base_e + base) * 16, _K * 16)], w_v)

                    def srow(k, _):
                        wrow = w_v[pl.ds(pl.multiple_of(k * 16, 16), 16)]
                        wk = jnp.full((16,), wrow[h], jnp.float32)
                        for f in range(8):
                            rows_v[k, pl.ds(f * 16, 16)] = (
                                rows_v[k, pl.ds(f * 16, 16)] * wk)
                        return 0

                    lax.fori_loop(0, _K, srow, 0)
                pltpu.sync_copy(rows_v, acc.at[didx_v], add=True)
                return 0

            lax.fori_loop(0, nchunk, chunk, 0)
            plsc.subcore_barrier()
            _writeout_acc(acc, lambda s, r: out.at[cid, h, pl.ds(s, r)],
                          sid, n)

    fn = pl.kernel(
        body,
        out_type=jax.ShapeDtypeStruct((_NCORES, heads, n, 128), jnp.float32),
        mesh=_mesh(),
        scratch_types=[
            pltpu.VMEM((ept,), jnp.int32),
            pltpu.VMEM((ept,), jnp.int32),
            pltpu.VMEM((_K,), jnp.int32),
            pltpu.VMEM((_K,), jnp.int32),
            pltpu.VMEM((_K, 128), jnp.float32),
            pltpu.VMEM((_K, 128), jnp.float32),
            pltpu.VMEM((_K * 16,), jnp.float32),
            pltpu.VMEM_SHARED((n, 128), jnp.float32),
        ],
    )
    return fn


def _sc_scores(n, e):
    ept = e // _NW
    nchunk = ept // _K

    def body(elr, src, dst, w_out, src_v, dst_v, sidx_v, didx_v, esrc_v,
             edst_v, wflat_v):
        cid = lax.axis_index("c")
        sid = lax.axis_index("s")
        base_e = (cid * _NSUB + sid) * ept
        pltpu.sync_copy(src.at[pl.ds(base_e, ept)], src_v)
        pltpu.sync_copy(dst.at[pl.ds(base_e, ept)], dst_v)

        def chunk(j, _):
            base = j * _K
            for g in range(_K // 16):
                sidx_v[pl.ds(g * 16, 16)] = src_v[pl.ds(base + g * 16, 16)]
                didx_v[pl.ds(g * 16, 16)] = dst_v[pl.ds(base + g * 16, 16)]
            pltpu.sync_copy(elr.at[sidx_v], esrc_v)
            pltpu.sync_copy(elr.at[didx_v], edst_v)

            def erow(k, _):
                s = esrc_v[k, pl.ds(0, 16)] + edst_v[k, pl.ds(8, 16)]
                s = jnp.maximum(s, 0.2 * s)
                wv = jnp.exp(s)
                wflat_v[pl.ds(pl.multiple_of(k * 16, 16), 16)] = wv
                return 0

            lax.fori_loop(0, _K, erow, 0)
            pltpu.sync_copy(wflat_v,
                            w_out.at[pl.ds((base_e + base) * 16, _K * 16)])
            return 0

        lax.fori_loop(0, nchunk, chunk, 0)

    fn = pl.kernel(
        body,
        out_type=jax.ShapeDtypeStruct((e * 16,), jnp.float32),
        mesh=_mesh(),
        scratch_types=[
            pltpu.VMEM((ept,), jnp.int32),
            pltpu.VMEM((ept,), jnp.int32),
            pltpu.VMEM((_K,), jnp.int32),
            pltpu.VMEM((_K,), jnp.int32),
            pltpu.VMEM((_K, 128), jnp.float32),
            pltpu.VMEM((_K, 128), jnp.float32),
            pltpu.VMEM((_K * 16,), jnp.float32),
        ],
    )
    return fn


def _sc_den(n, e):
    ept = e // _NW
    nchunk = ept // _K

    def body(w, dst, den_out, dst_v, didx_v, wflat_v, dbuf_v, zbuf_v, accd):
        cid = lax.axis_index("c")
        sid = lax.axis_index("s")
        base_e = (cid * _NSUB + sid) * ept
        pltpu.sync_copy(dst.at[pl.ds(base_e, ept)], dst_v)
        _zero_buf(zbuf_v, _K, 128)
        _zero_buf(dbuf_v, _K, 128)
        _zero_acc(zbuf_v, accd, sid, n)
        plsc.subcore_barrier()

        def chunk(j, _):
            base = j * _K
            for g in range(_K // 16):
                didx_v[pl.ds(g * 16, 16)] = dst_v[pl.ds(base + g * 16, 16)]
            pltpu.sync_copy(w.at[pl.ds((base_e + base) * 16, _K * 16)],
                            wflat_v)

            def erow(k, _):
                dbuf_v[k, pl.ds(0, 16)] = (
                    wflat_v[pl.ds(pl.multiple_of(k * 16, 16), 16)])
                return 0

            lax.fori_loop(0, _K, erow, 0)
            pltpu.sync_copy(dbuf_v, accd.at[didx_v], add=True)
            return 0

        lax.fori_loop(0, nchunk, chunk, 0)
        plsc.subcore_barrier()
        _writeout_acc(accd, lambda s, r: den_out.at[cid, pl.ds(s, r)],
                      sid, n)

    fn = pl.kernel(
        body,
        out_type=jax.ShapeDtypeStruct((_NCORES, n, 128), jnp.float32),
        mesh=_mesh(),
        scratch_types=[
            pltpu.VMEM((ept,), jnp.int32),
            pltpu.VMEM((_K,), jnp.int32),
            pltpu.VMEM((_K * 16,), jnp.float32),
            pltpu.VMEM((_K, 128), jnp.float32),
            pltpu.VMEM((_K, 128), jnp.float32),
            pltpu.VMEM_SHARED((n, 128), jnp.float32),
        ],
    )
    return fn


_TN = 1000


def _tc_layer1(n):
    nt = n // _TN

    def body(hn_b, hnp_b, hep_b, w1a_b, w1b_b, w1c_b, aelr_b, z_b, elr_b):
        h = pl.program_id(1)
        hnagg = hnp_b[0, 0] + hnp_b[1, 0]
        hea = hep_b[0, 0, :, 0:1] + hep_b[1, 0, :, 0:1]
        z = (jnp.dot(hn_b[...], w1a_b[...],
                     preferred_element_type=jnp.float32)
             + jnp.dot(hnagg, w1b_b[...],
                       preferred_element_type=jnp.float32)
             + hea * w1c_b[...])
        z_b[0] = z

        @pl.when(h == 0)
        def _():
            elr_b[...] = jnp.zeros_like(elr_b)

        elr_b[...] += jnp.dot(z, aelr_b[0],
                              preferred_element_type=jnp.float32)

    return pl.pallas_call(
        body,
        grid=(nt, 8),
        in_specs=[
            pl.BlockSpec((_TN, 128), lambda i, h: (i, 0)),
            pl.BlockSpec((2, 1, _TN, 128), lambda i, h: (0, 0, i, 0)),
            pl.BlockSpec((2, 1, _TN, 128), lambda i, h: (0, 0, i, 0)),
            pl.BlockSpec((128, 128), lambda i, h: (0, h)),
            pl.BlockSpec((128, 128), lambda i, h: (0, h)),
            pl.BlockSpec((1, 128), lambda i, h: (0, h)),
            pl.BlockSpec((1, 128, 128), lambda i, h: (h, 0, 0)),
        ],
        out_specs=[
            pl.BlockSpec((1, _TN, 128), lambda i, h: (h, i, 0)),
            pl.BlockSpec((_TN, 128), lambda i, h: (i, 0)),
        ],
        out_shape=[
            jax.ShapeDtypeStruct((8, n, 128), jnp.float32),
            jax.ShapeDtypeStruct((n, 128), jnp.float32),
        ],
    )


def _tc_gat_next(n, heads_out):
    nt = n // _TN

    def body(p_b, den_b, b_b, wr_b, aelr_b, z_b, elr_b, h1_s):
        hp = pl.program_id(1)

        @pl.when(hp == 0)
        def _():
            dsum = den_b[0, :, 0:8] + den_b[1, :, 0:8]
            for h in range(8):
                dcol = dsum[:, h:h + 1]
                dcol = jnp.where(dcol == 0.0, 1.0, dcol)
                h1_s[h] = jnp.maximum(
                    (p_b[0, h] + p_b[1, h]) / dcol + b_b[h][None, :], 0.0)

        z = jnp.zeros((_TN, 128), jnp.float32)
        for h in range(8):
            z = z + jnp.dot(h1_s[h], wr_b[0, h * 128:(h + 1) * 128, :],
                            preferred_element_type=jnp.float32)
        z_b[0] = z

        @pl.when(hp == 0)
        def _():
            elr_b[...] = jnp.zeros_like(elr_b)

        elr_b[...] += jnp.dot(z, aelr_b[0],
                              preferred_element_type=jnp.float32)

    return pl.pallas_call(
        body,
        grid=(nt, heads_out),
        in_specs=[
            pl.BlockSpec((2, 8, _TN, 128), lambda i, h: (0, 0, i, 0)),
            pl.BlockSpec((2, _TN, 128), lambda i, h: (0, i, 0)),
            pl.BlockSpec((8, 128), lambda i, h: (0, 0)),
            pl.BlockSpec((1, 1024, 128), lambda i, h: (h, 0, 0)),
            pl.BlockSpec((1, 128, 128), lambda i, h: (h, 0, 0)),
        ],
        out_specs=[
            pl.BlockSpec((1, _TN, 128), lambda i, h: (h, i, 0)),
            pl.BlockSpec((_TN, 128), lambda i, h: (i, 0)),
        ],
        out_shape=[
            jax.ShapeDtypeStruct((heads_out, n, 128), jnp.float32),
            jax.ShapeDtypeStruct((n, 128), jnp.float32),
        ],
        scratch_shapes=[pltpu.VMEM((8, _TN, 128), jnp.float32)],
    )


def _tc_gat3_out(n):
    nt = n // _TN

    def body(p_b, den_b, b_b, wb_b, h3_b, t3_b):
        dcol = den_b[0, :, 0:1] + den_b[1, :, 0:1]
        dcol = jnp.where(dcol == 0.0, 1.0, dcol)
        h3 = jnp.maximum((p_b[0, 0] + p_b[1, 0]) / dcol + b_b[...], 0.0)
        h3_b[...] = h3
        t3_b[...] = jnp.dot(h3, wb_b[...], preferred_element_type=jnp.float32)

    return pl.pallas_call(
        body,
        grid=(nt,),
        in_specs=[
            pl.BlockSpec((2, 1, _TN, 128), lambda i: (0, 0, i, 0)),
            pl.BlockSpec((2, _TN, 128), lambda i: (0, i, 0)),
            pl.BlockSpec((1, 128), lambda i: (0, 0)),
            pl.BlockSpec((128, 128), lambda i: (0, 0)),
        ],
        out_specs=[
            pl.BlockSpec((_TN, 128), lambda i: (i, 0)),
            pl.BlockSpec((_TN, 128), lambda i: (i, 0)),
        ],
        out_shape=[
            jax.ShapeDtypeStruct((n, 128), jnp.float32),
            jax.ShapeDtypeStruct((n, 128), jnp.float32),
        ],
    )


def _tc_sage(n):
    nt = n // _TN

    def body(h_b, agg_b, hep_b, wa_b, wc_b, b_b, wbn_b, hn_b, t_b):
        hea = hep_b[0, 0, :, 0:1] + hep_b[1, 0, :, 0:1]
        agg = agg_b[0, 0] + agg_b[1, 0]
        hnew = jnp.maximum(
            jnp.dot(h_b[...], wa_b[...], preferred_element_type=jnp.float32)
            + agg + hea * wc_b[...] + b_b[...], 0.0)
        hn_b[...] = hnew
        t_b[...] = jnp.dot(hnew, wbn_b[...],
                           preferred_element_type=jnp.float32)

    return pl.pallas_call(
        body,
        grid=(nt,),
        in_specs=[
            pl.BlockSpec((_TN, 128), lambda i: (i, 0)),
            pl.BlockSpec((2, 1, _TN, 128), lambda i: (0, 0, i, 0)),
            pl.BlockSpec((2, 1, _TN, 128), lambda i: (0, 0, i, 0)),
            pl.BlockSpec((128, 128), lambda i: (0, 0)),
            pl.BlockSpec((1, 128), lambda i: (0, 0)),
            pl.BlockSpec((1, 128), lambda i: (0, 0)),
            pl.BlockSpec((128, 128), lambda i: (0, 0)),
        ],
        out_specs=[
            pl.BlockSpec((_TN, 128), lambda i: (i, 0)),
            pl.BlockSpec((_TN, 128), lambda i: (i, 0)),
        ],
        out_shape=[
            jax.ShapeDtypeStruct((n, 128), jnp.float32),
            jax.ShapeDtypeStruct((n, 128), jnp.float32),
        ],
    )


def _tc_final(n):
    nt = n // _TN

    def body(h_b, agg_b, hep_b, wca_b, wcs_b, bc_b, o_b):
        hea = hep_b[0, 0, :, 0:1] + hep_b[1, 0, :, 0:1]
        agg = agg_b[0, 0, :, 0:1] + agg_b[1, 0, :, 0:1]
        y = jnp.dot(h_b[...], wca_b[...], preferred_element_type=jnp.float32)
        o_b[...] = y[:, 0:1] + agg + hea * wcs_b[0, 0] + bc_b[0, 0]

    return pl.pallas_call(
        body,
        grid=(nt,),
        in_specs=[
            pl.BlockSpec((_TN, 128), lambda i: (i, 0)),
            pl.BlockSpec((2, 1, _TN, 128), lambda i: (0, 0, i, 0)),
            pl.BlockSpec((2, 1, _TN, 128), lambda i: (0, 0, i, 0)),
            pl.BlockSpec((128, 128), lambda i: (0, 0)),
            pl.BlockSpec((1, 1), lambda i: (0, 0)),
            pl.BlockSpec((1, 1), lambda i: (0, 0)),
        ],
        out_specs=pl.BlockSpec((_TN, 1), lambda i: (i, 0)),
        out_shape=jax.ShapeDtypeStruct((n, 1), jnp.float32),
    )


def kernel(hn, he, edge_index, W1, al1, ar1, b1, W2, al2, ar2, b2,
           W3, al3, ar3, b3, Wm3, bm3, Wm4, bm4, Wm5, bm5, Wc, bc):
    n, d = hn.shape
    e = he.shape[0]
    f32 = jnp.float32
    src = edge_index[0]
    dst = edge_index[1]

    W1a, W1b, W1c = W1[:d], W1[d:2 * d], W1[2 * d:2 * d + 1]
    def _amat(al, ar):
        hds = al.shape[0]
        a = jnp.zeros((hds, 128, 128), f32)
        for h in range(hds):
            a = a.at[h, :, h].set(al[h])
            a = a.at[h, :, 8 + h].set(ar[h])
        return a

    AELR1 = _amat(al1, ar1)
    AELR2 = _amat(al2, ar2)
    AELR3 = _amat(al3, ar3)
    W2r = W2.reshape(1024, 8, 128).transpose(1, 0, 2)
    W3r = W3.reshape(1024, 1, 128).transpose(1, 0, 2)
    b1r, b2r = b1.reshape(8, 128), b2.reshape(8, 128)
    b3r = b3.reshape(1, 128)
    Wm3a, Wm3b, wm3c = Wm3[:128], Wm3[128:256], Wm3[256:257]
    Wm4a, Wm4b, wm4c = Wm4[:128], Wm4[128:256], Wm4[256:257]
    Wm5a, Wm5b, wm5c = Wm5[:128], Wm5[128:256], Wm5[256:257]
    Wca = jnp.pad(Wc[:128, 0:1], ((0, 0), (0, 127)))
    Wcb = jnp.pad(Wc[128:256, 0:1], ((0, 0), (0, 127)))
    wcs = Wc[256:257, 0:1]
    bcr = bc.reshape(1, 1)
    bm3r, bm4r, bm5r = bm3.reshape(1, 128), bm4.reshape(1, 128), bm5.reshape(1, 128)
    he16 = jnp.pad(he, ((0, 0), (0, 15)))

    sc_ones = _sc_gather_scatter(n, e, 1, scaled=False)
    sc_w8 = _sc_gather_scatter(n, e, 8, scaled=True)
    sc_w1 = _sc_gather_scatter(n, e, 1, scaled=True)
    sc_sc = _sc_scores(n, e)
    sc_den = _sc_den(n, e)
    dummy_w = jnp.zeros((8,), f32)
    ones_tab = jnp.ones((n, 128), f32)

    _PROBE = 0
    if _PROBE:
        def _rgat(x, W, al, ar, b, heads, od):
            z = (x @ W).reshape(n, heads, od)
            el = (z * al[None]).sum(-1)
            er = (z * ar[None]).sum(-1)
            ee = jax.nn.leaky_relu(el[src] + er[dst], negative_slope=0.2)
            emax = jax.ops.segment_max(ee, dst, num_segments=n)
            emax = jnp.where(jnp.isfinite(emax), emax, 0.0)
            eex = jnp.exp(ee - emax[dst])
            den = jax.ops.segment_sum(eex, dst, num_segments=n)[dst]
            alpha = eex / jnp.where(den == 0.0, 1.0, den)
            o = jax.ops.segment_sum(z[src] * alpha[..., None], dst,
                                    num_segments=n)
            return o + b.reshape(1, heads, od)

        def _rsage(h, W, b):
            hna = jax.ops.segment_sum(h[src], dst, num_segments=n)
            hea_ = jax.ops.segment_sum(he, dst, num_segments=n)
            return jnp.concatenate([h, hna, hea_], axis=1) @ W + b

        hna0 = jax.ops.segment_sum(hn[src], dst, num_segments=n)
        hea0 = jax.ops.segment_sum(he, dst, num_segments=n)
        ht = jnp.concatenate([hn, hna0, hea0], axis=1)
        rh = _rgat(ht, W1, al1, ar1, b1, 8, 128).reshape(n, -1)
        rh = jax.nn.relu(rh)
        rh = jax.nn.relu(_rgat(rh, W2, al2, ar2, b2, 8, 128).reshape(n, -1))
        rh = jax.nn.relu(_rgat(rh, W3, al3, ar3, b3, 1, 128).reshape(n, -1))
        rh = jax.nn.relu(_rsage(rh, Wm3, bm3))
        rh = jax.nn.relu(_rsage(rh, Wm4, bm4))
        rh = jax.nn.relu(_rsage(rh, Wm5, bm5))
        refout = _rsage(rh, Wc, bc)

        if _PROBE == 1:
            hep = sc_w1(ones_tab, dst, dst, he16.reshape(e * 16))
            got = (hep[0, 0] + hep[1, 0])[:, 0:1]
            return refout + (got - hea0) * 1e4
        if _PROBE == 2:
            p = sc_ones(hn, src, dst, dummy_w)
            got = p[0, 0] + p[1, 0]
            err = jnp.max(jnp.abs(got - hna0), axis=1, keepdims=True)
            return refout + err * 1e4
        if _PROBE == 3:
            wtest = jnp.abs(jnp.sin(jnp.arange(e * 16, dtype=f32)))
            p = sc_w8(jnp.tile(hn, (8, 1)), src, dst, wtest)
            wr = wtest.reshape(e, 16)
            err = jnp.zeros((n, 1), f32)
            for hh in (0, 7):
                exph = jax.ops.segment_sum(hn[src] * wr[:, hh:hh + 1], dst,
                                           num_segments=n)
                goth = p[0, hh] + p[1, hh]
                err = err + jnp.max(jnp.abs(goth - exph), axis=1,
                                    keepdims=True)
            return refout + err * 1e4
        if _PROBE == 4:
            z1j = (ht @ W1).reshape(n, 8, 128)
            elj = (z1j * al1[None]).sum(-1)
            erj = (z1j * ar1[None]).sum(-1)
            elrj = jnp.concatenate(
                [elj, erj, jnp.zeros((n, 112), f32)], axis=1)
            wj = sc_sc(elrj, src, dst)
            wj = lax.optimization_barrier(wj)
            denp = sc_den(wj, dst)
            eej = jax.nn.leaky_relu(elj[src] + erj[dst], 0.2)
            wexp = jnp.exp(eej)
            werr = jax.ops.segment_sum(
                jnp.max(jnp.abs(wj.reshape(e, 16)[:, 0:8] - wexp), axis=1),
                dst, num_segments=n)[:, None]
            denj = jax.ops.segment_sum(wexp, dst, num_segments=n)
            derr = jnp.max(jnp.abs((denp[0] + denp[1])[:, 0:8] - denj),
                           axis=1, keepdims=True)
            return refout + derr * 1e4
        if _PROBE == 6:
            z1j = (ht @ W1).reshape(n, 8, 128)
            elj = (z1j * al1[None]).sum(-1)
            erj = (z1j * ar1[None]).sum(-1)
            elrj = jnp.concatenate(
                [elj, erj, jnp.zeros((n, 112), f32)], axis=1)
            wj = sc_sc(elrj, src, dst)
            eej = jax.nn.leaky_relu(elj[src] + erj[dst], 0.2)
            wexp = jnp.exp(eej)
            wexp16 = jnp.pad(wexp, ((0, 0), (0, 8))).reshape(e * 16)
            denp = sc_den(wexp16 + 0.0 * wj[0], dst)
            denj = jax.ops.segment_sum(wexp, dst, num_segments=n)
            derr6 = jnp.max(jnp.abs((denp[0] + denp[1])[:, 0:8] - denj),
                            axis=1, keepdims=True)
            return refout + derr6 * 1e4
        if _PROBE == 5:
            denp = sc_den(he16.reshape(e * 16), dst)
            derr5 = jnp.abs((denp[0] + denp[1])[:, 0:1] - hea0)
            return refout + derr5 * 1e4

    hep = sc_w1(ones_tab, dst, dst, he16.reshape(e * 16))
    hnaggp = sc_ones(_chain(hn, hep), src, dst, dummy_w)
    z1, elr1 = _tc_layer1(n)(hn, hnaggp, hep, W1a, W1b, W1c, AELR1)
    w1 = sc_sc(_chain(elr1, hnaggp), src, dst)
    den1p = sc_den(w1, dst)
    out1p = sc_w8(_chain(z1.reshape(8 * n, 128), den1p), src, dst, w1)
    z2, elr2 = _tc_gat_next(n, 8)(out1p, den1p, b1r, W2r, AELR2)
    w2 = sc_sc(_chain(elr2, out1p), src, dst)
    den2p = sc_den(w2, dst)
    out2p = sc_w8(_chain(z2.reshape(8 * n, 128), den2p), src, dst, w2)
    z3, elr3 = _tc_gat_next(n, 1)(out2p, den2p, b2r, W3r, AELR3)
    w3 = sc_sc(_chain(elr3, out2p), src, dst)
    den3p = sc_den(w3, dst)
    out3p = sc_w1(_chain(z3.reshape(n, 128), den3p), src, dst, w3)
    h3, t3 = _tc_gat3_out(n)(out3p, den3p, b3r, Wm3b)
    ag3p = sc_ones(_chain(t3, out3p), src, dst, dummy_w)
    h4, t4 = _tc_sage(n)(h3, ag3p, hep, Wm3a, wm3c, bm3r, Wm4b)
    ag4p = sc_ones(_chain(t4, ag3p), src, dst, dummy_w)
    h5, t5 = _tc_sage(n)(h4, ag4p, hep, Wm4a, wm4c, bm4r, Wm5b)
    ag5p = sc_ones(_chain(t5, ag4p), src, dst, dummy_w)
    h6, tc = _tc_sage(n)(h5, ag5p, hep, Wm5a, wm5c, bm5r, Wcb)
    agcp = sc_ones(_chain(tc, ag5p), src, dst, dummy_w)
    out = _tc_final(n)(h6, agcp, hep, Wca, wcs, bcr)
    return out
